# baseline (speedup 1.0000x reference)
import numpy as np
import jax
import jax.numpy as jnp
from jax.experimental import pallas as pl
from jax.experimental.pallas import tpu as pltpu

_C = 96
_H = 224
_W = 224
_R = 32
_NG = 32
_GS = _C // _NG


def _resize_composite(n: int) -> np.ndarray:
    half = n // 2
    d = np.zeros((half, n), np.float32)
    d[np.arange(half), 2 * np.arange(half)] = 0.5
    d[np.arange(half), 2 * np.arange(half) + 1] = 0.5
    u = np.zeros((n, half), np.float32)
    for i in range(n):
        p = (i + 0.5) * 0.5 - 0.5
        f = int(np.floor(p))
        fr = p - f
        i0 = min(max(f, 0), half - 1)
        i1 = min(max(f + 1, 0), half - 1)
        u[i, i0] += 1.0 - fr
        u[i, i1] += fr
    return (u @ d).astype(np.float32)


_RESIZE_M = _resize_composite(_H)

_GMAT = np.zeros((_C, _C), np.float32)
for _c in range(_C):
    _g = _c // _GS
    _GMAT[_c, _g * _GS:(_g + 1) * _GS] = 1.0 / (_GS * _H * _W)


_CB = 8


def _df_kernel(x_ref, m_ref, mt_ref, df_ref, s1_ref, s2_ref):
    k = pl.program_id(0)
    xc8 = x_ref[...]
    s1_ref[...] = xc8.sum(axis=2).sum(axis=1, keepdims=True)
    s2_ref[...] = (xc8 * xc8).sum(axis=2).sum(axis=1, keepdims=True)
    mm = m_ref[...]
    mt = mt_ref[...]
    part = None
    for c in range(_CB):
        xc = xc8[c]
        u = jnp.dot(jnp.dot(mm, xc, preferred_element_type=jnp.float32), mt,
                    preferred_element_type=jnp.float32)
        a = jnp.abs(xc - u)
        part = a if part is None else part + a

    @pl.when(k == 0)
    def _():
        df_ref[...] = part

    @pl.when(k > 0)
    def _():
        df_ref[...] += part


def _main_kernel(x0_ref, x1_ref, x2_ref, df_ref, dfa_ref, s1_ref,
                 s2_ref, g_ref, gnw_ref, gnb_ref, w1_ref, b1_ref, w2_ref,
                 b2_ref, out_ref):
    xr = (x0_ref, x1_ref, x2_ref)

    s1 = s1_ref[...]
    s2 = s2_ref[...]
    mean = jnp.dot(g_ref[...], s1, preferred_element_type=jnp.float32)
    ex2 = jnp.dot(g_ref[...], s2, preferred_element_type=jnp.float32)
    var = ex2 - mean * mean
    scale = gnw_ref[...] / jnp.sqrt(var + 1e-5)
    shift = gnb_ref[...] - mean * scale

    dfall = dfa_ref[...]
    dmin = jnp.min(dfall)
    dmax = jnp.max(dfall)
    dft = df_ref[...]
    dn = (dft - dmin) / (dmax - dmin + 1e-8)
    dp = dn * dn
    thr = (dp > 0.1).astype(jnp.float32)
    conn = 1.0 + jnp.maximum(jnp.round(dp * 15.0), 0.0) * thr
    k_i = jnp.minimum(conn.astype(jnp.int32), 9)
    validm = conn > 1.0

    n2a = jnp.sum(x0_ref[...] * x0_ref[...], axis=0)
    xb = x2_ref[:, _R - 2:, :]
    n2b = jnp.sum(xb * xb, axis=0)
    n2 = jnp.concatenate([n2a, n2b], axis=0)
    inv = 1.0 / jnp.maximum(jnp.sqrt(n2), 1e-12)
    invc = inv[1:_R + 1, 1:_W + 1]

    cc = x1_ref[:, :, 1:_W + 1]
    sims = []
    for d in range(9):
        di, dj = d // 3, d % 3
        nb = cc if d == 4 else xr[di][:, :, dj:dj + _W]
        dot = jnp.sum(cc * nb, axis=0)
        sims.append(dot * invc * inv[di:di + _R, dj:dj + _W])

    weights = []
    wsum = None
    for j in range(9):
        rank = jnp.zeros((_R, _W), jnp.int32)
        for t in range(9):
            if t == j:
                continue
            if t < j:
                cond = sims[t] >= sims[j]
            else:
                cond = sims[t] > sims[j]
            rank = rank + cond.astype(jnp.int32)
        sel = (rank < k_i).astype(jnp.float32)
        wj = jnp.exp(sims[j]) * sel
        weights.append(wj)
        wsum = wj if wsum is None else wsum + wj

    winv = 1.0 / wsum
    acc = None
    for d in range(9):
        di, dj = d // 3, d % 3
        nb = cc if d == 4 else xr[di][:, :, dj:dj + _W]
        term = (weights[d] * winv)[None, :, :] * nb
        acc = term if acc is None else acc + term

    outm = jnp.where(validm[None, :, :], acc, cc)
    e = outm + cc * scale[:, :, None] + shift[:, :, None]
    e2 = e.reshape(_C, _R * _W)
    h = jnp.maximum(
        jnp.dot(w1_ref[...], e2, preferred_element_type=jnp.float32)
        + b1_ref[...], 0.0)
    f = e2 + jnp.dot(w2_ref[...], h, preferred_element_type=jnp.float32) \
        + b2_ref[...]
    out_ref[...] = f.reshape(_C, _R, _W)


def kernel(x, gn_weight, gn_bias, w1, b1, w2, b2):
    x3 = x[0]
    xd0 = jnp.pad(x3[:, :_H - 1, :], ((0, 0), (1, 0), (1, 1)))
    xd1 = jnp.pad(x3, ((0, 0), (0, 0), (1, 1)))
    xd2 = jnp.pad(x3[:, 1:, :], ((0, 0), (0, 1), (1, 1)))
    m = jnp.asarray(_RESIZE_M)
    mt = jnp.asarray(_RESIZE_M.T)
    gmat = jnp.asarray(_GMAT)

    df, s1, s2 = pl.pallas_call(
        _df_kernel,
        grid=(_C // _CB,),
        in_specs=[
            pl.BlockSpec((_CB, _H, _W), lambda k: (k, 0, 0)),
            pl.BlockSpec((_H, _H), lambda k: (0, 0)),
            pl.BlockSpec((_H, _H), lambda k: (0, 0)),
        ],
        out_specs=[
            pl.BlockSpec((_H, _W), lambda k: (0, 0)),
            pl.BlockSpec((_CB, 1), lambda k: (k, 0)),
            pl.BlockSpec((_CB, 1), lambda k: (k, 0)),
        ],
        out_shape=[
            jax.ShapeDtypeStruct((_H, _W), jnp.float32),
            jax.ShapeDtypeStruct((_C, 1), jnp.float32),
            jax.ShapeDtypeStruct((_C, 1), jnp.float32),
        ],
    )(x3, m, mt)

    out = pl.pallas_call(
        _main_kernel,
        grid=(_H // _R,),
        in_specs=[
            pl.BlockSpec((_C, _R, _W + 2), lambda i: (0, i, 0)),
            pl.BlockSpec((_C, _R, _W + 2), lambda i: (0, i, 0)),
            pl.BlockSpec((_C, _R, _W + 2), lambda i: (0, i, 0)),
            pl.BlockSpec((_R, _W), lambda i: (i, 0)),
            pl.BlockSpec((_H, _W), lambda i: (0, 0)),
            pl.BlockSpec((_C, 1), lambda i: (0, 0)),
            pl.BlockSpec((_C, 1), lambda i: (0, 0)),
            pl.BlockSpec((_C, _C), lambda i: (0, 0)),
            pl.BlockSpec((_C, 1), lambda i: (0, 0)),
            pl.BlockSpec((_C, 1), lambda i: (0, 0)),
            pl.BlockSpec((2 * _C, _C), lambda i: (0, 0)),
            pl.BlockSpec((2 * _C, 1), lambda i: (0, 0)),
            pl.BlockSpec((_C, 2 * _C), lambda i: (0, 0)),
            pl.BlockSpec((_C, 1), lambda i: (0, 0)),
        ],
        out_specs=pl.BlockSpec((_C, _R, _W), lambda i: (0, i, 0)),
        out_shape=jax.ShapeDtypeStruct((_C, _H, _W), jnp.float32),
        compiler_params=pltpu.CompilerParams(vmem_limit_bytes=110 * 2**20),
    )(xd0, xd1, xd2, df, df, s1, s2,
      gmat, gn_weight[:, None], gn_bias[:, None], w1, b1[:, None], w2,
      b2[:, None])

    return out[None]


# speedup vs baseline: 1.0149x; 1.0149x over previous
import numpy as np
import jax
import jax.numpy as jnp
from jax.experimental import pallas as pl
from jax.experimental.pallas import tpu as pltpu

_C = 96
_H = 224
_W = 224
_R = 16
_NG = 32
_GS = _C // _NG


def _resize_composite(n: int) -> np.ndarray:
    half = n // 2
    d = np.zeros((half, n), np.float32)
    d[np.arange(half), 2 * np.arange(half)] = 0.5
    d[np.arange(half), 2 * np.arange(half) + 1] = 0.5
    u = np.zeros((n, half), np.float32)
    for i in range(n):
        p = (i + 0.5) * 0.5 - 0.5
        f = int(np.floor(p))
        fr = p - f
        i0 = min(max(f, 0), half - 1)
        i1 = min(max(f + 1, 0), half - 1)
        u[i, i0] += 1.0 - fr
        u[i, i1] += fr
    return (u @ d).astype(np.float32)


_RESIZE_M = _resize_composite(_H)

_GMAT = np.zeros((_C, _C), np.float32)
for _c in range(_C):
    _g = _c // _GS
    _GMAT[_c, _g * _GS:(_g + 1) * _GS] = 1.0 / (_GS * _H * _W)


_CB = 8


def _df_kernel(x_ref, m_ref, mt_ref, df_ref, s1_ref, s2_ref):
    k = pl.program_id(0)
    xc8 = x_ref[...]
    s1_ref[...] = xc8.sum(axis=2).sum(axis=1, keepdims=True)
    s2_ref[...] = (xc8 * xc8).sum(axis=2).sum(axis=1, keepdims=True)
    mm = m_ref[...]
    mt = mt_ref[...]
    part = None
    for c in range(_CB):
        xc = xc8[c]
        u = jnp.dot(jnp.dot(mm, xc, preferred_element_type=jnp.float32), mt,
                    preferred_element_type=jnp.float32)
        a = jnp.abs(xc - u)
        part = a if part is None else part + a

    @pl.when(k == 0)
    def _():
        df_ref[...] = part

    @pl.when(k > 0)
    def _():
        df_ref[...] += part


def _main_kernel(x0_ref, x1_ref, x2_ref, df_ref, dfa_ref, s1_ref,
                 s2_ref, g_ref, gnw_ref, gnb_ref, w1_ref, b1_ref, w2_ref,
                 b2_ref, out_ref):
    xr = (x0_ref, x1_ref, x2_ref)

    s1 = s1_ref[...]
    s2 = s2_ref[...]
    mean = jnp.dot(g_ref[...], s1, preferred_element_type=jnp.float32)
    ex2 = jnp.dot(g_ref[...], s2, preferred_element_type=jnp.float32)
    var = ex2 - mean * mean
    scale = gnw_ref[...] / jnp.sqrt(var + 1e-5)
    shift = gnb_ref[...] - mean * scale

    dfall = dfa_ref[...]
    dmin = jnp.min(dfall)
    dmax = jnp.max(dfall)
    dft = df_ref[...]
    dn = (dft - dmin) / (dmax - dmin + 1e-8)
    dp = dn * dn
    thr = (dp > 0.1).astype(jnp.float32)
    conn = 1.0 + jnp.maximum(jnp.round(dp * 15.0), 0.0) * thr
    k_i = jnp.minimum(conn.astype(jnp.int32), 9)
    validm = conn > 1.0

    n2a = jnp.sum(x0_ref[...] * x0_ref[...], axis=0)
    xb = x2_ref[:, _R - 2:, :]
    n2b = jnp.sum(xb * xb, axis=0)
    n2 = jnp.concatenate([n2a, n2b], axis=0)
    inv = 1.0 / jnp.maximum(jnp.sqrt(n2), 1e-12)
    invc = inv[1:_R + 1, 1:_W + 1]

    cc = x1_ref[:, :, 1:_W + 1]
    sims = []
    for d in range(9):
        di, dj = d // 3, d % 3
        nb = cc if d == 4 else xr[di][:, :, dj:dj + _W]
        dot = jnp.sum(cc * nb, axis=0)
        sims.append(dot * invc * inv[di:di + _R, dj:dj + _W])

    weights = []
    wsum = None
    for j in range(9):
        rank = jnp.zeros((_R, _W), jnp.int32)
        for t in range(9):
            if t == j:
                continue
            if t < j:
                cond = sims[t] >= sims[j]
            else:
                cond = sims[t] > sims[j]
            rank = rank + cond.astype(jnp.int32)
        sel = (rank < k_i).astype(jnp.float32)
        wj = jnp.exp(sims[j]) * sel
        weights.append(wj)
        wsum = wj if wsum is None else wsum + wj

    winv = 1.0 / wsum
    acc = None
    for d in range(9):
        di, dj = d // 3, d % 3
        nb = cc if d == 4 else xr[di][:, :, dj:dj + _W]
        term = (weights[d] * winv)[None, :, :] * nb
        acc = term if acc is None else acc + term

    outm = jnp.where(validm[None, :, :], acc, cc)
    e = outm + cc * scale[:, :, None] + shift[:, :, None]
    e2 = e.reshape(_C, _R * _W)
    h = jnp.maximum(
        jnp.dot(w1_ref[...], e2, preferred_element_type=jnp.float32)
        + b1_ref[...], 0.0)
    f = e2 + jnp.dot(w2_ref[...], h, preferred_element_type=jnp.float32) \
        + b2_ref[...]
    out_ref[...] = f.reshape(_C, _R, _W)


def kernel(x, gn_weight, gn_bias, w1, b1, w2, b2):
    x3 = x[0]
    xd0 = jnp.pad(x3[:, :_H - 1, :], ((0, 0), (1, 0), (1, 1)))
    xd1 = jnp.pad(x3, ((0, 0), (0, 0), (1, 1)))
    xd2 = jnp.pad(x3[:, 1:, :], ((0, 0), (0, 1), (1, 1)))
    m = jnp.asarray(_RESIZE_M)
    mt = jnp.asarray(_RESIZE_M.T)
    gmat = jnp.asarray(_GMAT)

    df, s1, s2 = pl.pallas_call(
        _df_kernel,
        grid=(_C // _CB,),
        in_specs=[
            pl.BlockSpec((_CB, _H, _W), lambda k: (k, 0, 0)),
            pl.BlockSpec((_H, _H), lambda k: (0, 0)),
            pl.BlockSpec((_H, _H), lambda k: (0, 0)),
        ],
        out_specs=[
            pl.BlockSpec((_H, _W), lambda k: (0, 0)),
            pl.BlockSpec((_CB, 1), lambda k: (k, 0)),
            pl.BlockSpec((_CB, 1), lambda k: (k, 0)),
        ],
        out_shape=[
            jax.ShapeDtypeStruct((_H, _W), jnp.float32),
            jax.ShapeDtypeStruct((_C, 1), jnp.float32),
            jax.ShapeDtypeStruct((_C, 1), jnp.float32),
        ],
    )(x3, m, mt)

    out = pl.pallas_call(
        _main_kernel,
        grid=(_H // _R,),
        in_specs=[
            pl.BlockSpec((_C, _R, _W + 2), lambda i: (0, i, 0)),
            pl.BlockSpec((_C, _R, _W + 2), lambda i: (0, i, 0)),
            pl.BlockSpec((_C, _R, _W + 2), lambda i: (0, i, 0)),
            pl.BlockSpec((_R, _W), lambda i: (i, 0)),
            pl.BlockSpec((_H, _W), lambda i: (0, 0)),
            pl.BlockSpec((_C, 1), lambda i: (0, 0)),
            pl.BlockSpec((_C, 1), lambda i: (0, 0)),
            pl.BlockSpec((_C, _C), lambda i: (0, 0)),
            pl.BlockSpec((_C, 1), lambda i: (0, 0)),
            pl.BlockSpec((_C, 1), lambda i: (0, 0)),
            pl.BlockSpec((2 * _C, _C), lambda i: (0, 0)),
            pl.BlockSpec((2 * _C, 1), lambda i: (0, 0)),
            pl.BlockSpec((_C, 2 * _C), lambda i: (0, 0)),
            pl.BlockSpec((_C, 1), lambda i: (0, 0)),
        ],
        out_specs=pl.BlockSpec((_C, _R, _W), lambda i: (0, i, 0)),
        out_shape=jax.ShapeDtypeStruct((_C, _H, _W), jnp.float32),
        compiler_params=pltpu.CompilerParams(vmem_limit_bytes=110 * 2**20),
    )(xd0, xd1, xd2, df, df, s1, s2,
      gmat, gn_weight[:, None], gn_bias[:, None], w1, b1[:, None], w2,
      b2[:, None])

    return out[None]


# speedup vs baseline: 1.0150x; 1.0001x over previous
import numpy as np
import jax
import jax.numpy as jnp
from jax.experimental import pallas as pl
from jax.experimental.pallas import tpu as pltpu

_C = 96
_H = 224
_W = 224
_R = 16
_NG = 32
_GS = _C // _NG


def _resize_composite(n: int) -> np.ndarray:
    half = n // 2
    d = np.zeros((half, n), np.float32)
    d[np.arange(half), 2 * np.arange(half)] = 0.5
    d[np.arange(half), 2 * np.arange(half) + 1] = 0.5
    u = np.zeros((n, half), np.float32)
    for i in range(n):
        p = (i + 0.5) * 0.5 - 0.5
        f = int(np.floor(p))
        fr = p - f
        i0 = min(max(f, 0), half - 1)
        i1 = min(max(f + 1, 0), half - 1)
        u[i, i0] += 1.0 - fr
        u[i, i1] += fr
    return (u @ d).astype(np.float32)


_RESIZE_M = _resize_composite(_H)

_GMAT = np.zeros((_C, _C), np.float32)
for _c in range(_C):
    _g = _c // _GS
    _GMAT[_c, _g * _GS:(_g + 1) * _GS] = 1.0 / (_GS * _H * _W)


_CB = 8


def _df_kernel(x_ref, m_ref, mt_ref, df_ref, s1_ref, s2_ref):
    k = pl.program_id(0)
    xc8 = x_ref[...]
    s1_ref[...] = xc8.sum(axis=2).sum(axis=1, keepdims=True)
    s2_ref[...] = (xc8 * xc8).sum(axis=2).sum(axis=1, keepdims=True)
    mm = m_ref[...]
    mt = mt_ref[...]
    part = None
    for c in range(_CB):
        xc = xc8[c]
        u = jnp.dot(jnp.dot(mm, xc, preferred_element_type=jnp.float32), mt,
                    preferred_element_type=jnp.float32)
        a = jnp.abs(xc - u)
        part = a if part is None else part + a

    @pl.when(k == 0)
    def _():
        df_ref[...] = part

    @pl.when(k > 0)
    def _():
        df_ref[...] += part


def _main_kernel(x1_ref, ht_ref, hb_ref, df_ref, dfa_ref, s1_ref,
                 s2_ref, g_ref, gnw_ref, gnb_ref, w1_ref, b1_ref, w2_ref,
                 b2_ref, out_ref):
    blk = x1_ref[...]
    ht0 = ht_ref[0][:, None, :]
    hb0 = hb_ref[0][:, None, :]
    w_up = jnp.concatenate([ht0, blk[:, :_R - 1, :]], axis=1)
    w_dn = jnp.concatenate([blk[:, 1:, :], hb0], axis=1)
    xr = (w_up, blk, w_dn)

    s1 = s1_ref[...]
    s2 = s2_ref[...]
    mean = jnp.dot(g_ref[...], s1, preferred_element_type=jnp.float32)
    ex2 = jnp.dot(g_ref[...], s2, preferred_element_type=jnp.float32)
    var = ex2 - mean * mean
    scale = gnw_ref[...] / jnp.sqrt(var + 1e-5)
    shift = gnb_ref[...] - mean * scale

    dfall = dfa_ref[...]
    dmin = jnp.min(dfall)
    dmax = jnp.max(dfall)
    dft = df_ref[...]
    dn = (dft - dmin) / (dmax - dmin + 1e-8)
    dp = dn * dn
    thr = (dp > 0.1).astype(jnp.float32)
    conn = 1.0 + jnp.maximum(jnp.round(dp * 15.0), 0.0) * thr
    k_i = jnp.minimum(conn.astype(jnp.int32), 9)
    validm = conn > 1.0

    n2a = jnp.sum(w_up * w_up, axis=0)
    xb = w_dn[:, _R - 2:, :]
    n2b = jnp.sum(xb * xb, axis=0)
    n2 = jnp.concatenate([n2a, n2b], axis=0)
    inv = 1.0 / jnp.maximum(jnp.sqrt(n2), 1e-12)
    invc = inv[1:_R + 1, 1:_W + 1]

    cc = blk[:, :, 1:_W + 1]
    sims = []
    for d in range(9):
        di, dj = d // 3, d % 3
        nb = cc if d == 4 else xr[di][:, :, dj:dj + _W]
        dot = jnp.sum(cc * nb, axis=0)
        sims.append(dot * invc * inv[di:di + _R, dj:dj + _W])

    weights = []
    wsum = None
    for j in range(9):
        rank = jnp.zeros((_R, _W), jnp.int32)
        for t in range(9):
            if t == j:
                continue
            if t < j:
                cond = sims[t] >= sims[j]
            else:
                cond = sims[t] > sims[j]
            rank = rank + cond.astype(jnp.int32)
        sel = (rank < k_i).astype(jnp.float32)
        wj = jnp.exp(sims[j]) * sel
        weights.append(wj)
        wsum = wj if wsum is None else wsum + wj

    winv = 1.0 / wsum
    acc = None
    for d in range(9):
        di, dj = d // 3, d % 3
        nb = cc if d == 4 else xr[di][:, :, dj:dj + _W]
        term = (weights[d] * winv)[None, :, :] * nb
        acc = term if acc is None else acc + term

    outm = jnp.where(validm[None, :, :], acc, cc)
    e = outm + cc * scale[:, :, None] + shift[:, :, None]
    e2 = e.reshape(_C, _R * _W)
    h = jnp.maximum(
        jnp.dot(w1_ref[...], e2, preferred_element_type=jnp.float32)
        + b1_ref[...], 0.0)
    f = e2 + jnp.dot(w2_ref[...], h, preferred_element_type=jnp.float32) \
        + b2_ref[...]
    out_ref[...] = f.reshape(_C, _R, _W)


def kernel(x, gn_weight, gn_bias, w1, b1, w2, b2):
    x3 = x[0]
    xd = jnp.pad(x3, ((0, 0), (0, 0), (1, 1)))
    zrow = jnp.zeros((1, _C, _W + 2), jnp.float32)
    ht = jnp.concatenate(
        [zrow, jnp.transpose(xd[:, _R - 1:_H - _R:_R, :], (1, 0, 2))], axis=0)
    hb = jnp.concatenate(
        [jnp.transpose(xd[:, _R:_H - _R + 1:_R, :], (1, 0, 2)), zrow], axis=0)
    m = jnp.asarray(_RESIZE_M)
    mt = jnp.asarray(_RESIZE_M.T)
    gmat = jnp.asarray(_GMAT)

    df, s1, s2 = pl.pallas_call(
        _df_kernel,
        grid=(_C // _CB,),
        in_specs=[
            pl.BlockSpec((_CB, _H, _W), lambda k: (k, 0, 0)),
            pl.BlockSpec((_H, _H), lambda k: (0, 0)),
            pl.BlockSpec((_H, _H), lambda k: (0, 0)),
        ],
        out_specs=[
            pl.BlockSpec((_H, _W), lambda k: (0, 0)),
            pl.BlockSpec((_CB, 1), lambda k: (k, 0)),
            pl.BlockSpec((_CB, 1), lambda k: (k, 0)),
        ],
        out_shape=[
            jax.ShapeDtypeStruct((_H, _W), jnp.float32),
            jax.ShapeDtypeStruct((_C, 1), jnp.float32),
            jax.ShapeDtypeStruct((_C, 1), jnp.float32),
        ],
    )(x3, m, mt)

    out = pl.pallas_call(
        _main_kernel,
        grid=(_H // _R,),
        in_specs=[
            pl.BlockSpec((_C, _R, _W + 2), lambda i: (0, i, 0)),
            pl.BlockSpec((1, _C, _W + 2), lambda i: (i, 0, 0)),
            pl.BlockSpec((1, _C, _W + 2), lambda i: (i, 0, 0)),
            pl.BlockSpec((_R, _W), lambda i: (i, 0)),
            pl.BlockSpec((_H, _W), lambda i: (0, 0)),
            pl.BlockSpec((_C, 1), lambda i: (0, 0)),
            pl.BlockSpec((_C, 1), lambda i: (0, 0)),
            pl.BlockSpec((_C, _C), lambda i: (0, 0)),
            pl.BlockSpec((_C, 1), lambda i: (0, 0)),
            pl.BlockSpec((_C, 1), lambda i: (0, 0)),
            pl.BlockSpec((2 * _C, _C), lambda i: (0, 0)),
            pl.BlockSpec((2 * _C, 1), lambda i: (0, 0)),
            pl.BlockSpec((_C, 2 * _C), lambda i: (0, 0)),
            pl.BlockSpec((_C, 1), lambda i: (0, 0)),
        ],
        out_specs=pl.BlockSpec((_C, _R, _W), lambda i: (0, i, 0)),
        out_shape=jax.ShapeDtypeStruct((_C, _H, _W), jnp.float32),
        compiler_params=pltpu.CompilerParams(vmem_limit_bytes=110 * 2**20),
    )(xd, ht, hb, df, df, s1, s2,
      gmat, gn_weight[:, None], gn_bias[:, None], w1, b1[:, None], w2,
      b2[:, None])

    return out[None]
